# baseline (speedup 1.0000x reference)
import functools

import jax
import jax.numpy as jnp
import numpy as np
from jax import lax
from jax.experimental import pallas as pl
from jax.experimental.pallas import tpu as pltpu
from jax.experimental.pallas import tpu_sc as plsc

_B = 16
_ROW = 2048
_N = _B * _ROW
_LANES = 16
_HALF = _ROW // 2
_ITERS = _HALF // _LANES


def _deg2dist(deg):
    return 2.0 * np.sin(np.radians(deg) / 2.0)


_THRESH_DEG = np.ones(_B, dtype=np.float32)
_T_MIN = np.float32(_deg2dist(10.0 / 3600.0) ** 2)
_T_MAX = (_deg2dist(_THRESH_DEG) ** 2).astype(np.float32)
_LOG_RANGE = np.log(_T_MAX / _T_MIN).astype(np.float32)
_INV_T_MIN = float(1.0 / _T_MIN)


def _body(up_hbm, uo_hbm, nh_hbm, r_hbm, pp_hbm, lr_hbm,
          v_hbm, h_hbm, lam_hbm,
          ux_v, uy_v, uz_v, ox_v, oy_v, oz_v, v_v, m_v,
          nh_v, r_v, pp_v, lr_v, lam_v, cnt_v, pc_v, shared, sem):
    c = lax.axis_index("c")
    s = lax.axis_index("s")
    b = c * 8 + lax.div(s, 2)
    half = lax.rem(s, 2)
    base = b * _ROW + half * _HALF

    cps = [
        pltpu.async_copy(up_hbm.at[pl.ds(base, _HALF)], ux_v, sem),
        pltpu.async_copy(up_hbm.at[pl.ds(_N + base, _HALF)], uy_v, sem),
        pltpu.async_copy(up_hbm.at[pl.ds(2 * _N + base, _HALF)], uz_v, sem),
        pltpu.async_copy(uo_hbm.at[pl.ds(base, _HALF)], ox_v, sem),
        pltpu.async_copy(uo_hbm.at[pl.ds(_N + base, _HALF)], oy_v, sem),
        pltpu.async_copy(uo_hbm.at[pl.ds(2 * _N + base, _HALF)], oz_v, sem),
        pltpu.async_copy(nh_hbm, nh_v, sem),
        pltpu.async_copy(r_hbm, r_v, sem),
        pltpu.async_copy(pp_hbm, pp_v, sem),
        pltpu.async_copy(lr_hbm, lr_v, sem),
    ]
    for cp in cps:
        cp.wait()

    lane = lax.iota(jnp.int32, _LANES)
    sel = lane == b

    pl_ = pp_v[...] * lr_v[...]
    thresh_vec = _T_MIN * jnp.exp(pl_)
    inv_thresh_vec = _INV_T_MIN * jnp.exp(-pl_)
    thr = jnp.sum(jnp.where(sel, thresh_vec, 0.0))
    inv_thr = jnp.sum(jnp.where(sel, inv_thresh_vec, 0.0))
    nh = jnp.sum(jnp.where(sel, nh_v[...], 0.0))

    def step(j, cnt):
        k = j * _LANES
        sl = pl.ds(k, _LANES)
        dx = ux_v[sl] - ox_v[sl]
        dy = uy_v[sl] - oy_v[sl]
        dz = uz_v[sl] - oz_v[sl]
        s2 = dx * dx + dy * dy + dz * dz
        m = s2 < thr
        mf = jnp.where(m, 1.0, 0.0)
        v_v[sl] = jnp.where(m, s2 * inv_thr, 0.0)
        m_v[sl] = mf
        return cnt + mf

    cnt = lax.fori_loop(0, _ITERS, step, jnp.zeros((_LANES,), jnp.float32),
                        unroll=4)

    cnt_v[...] = cnt
    pltpu.sync_copy(cnt_v, shared.at[pl.ds(s * _LANES, _LANES)])
    plsc.subcore_barrier()
    partner = s + 1 - 2 * half
    pltpu.sync_copy(shared.at[pl.ds(partner * _LANES, _LANES)], pc_v)
    count = jnp.sum(cnt + pc_v[...])
    h16 = jnp.full((_LANES,), nh) / jnp.full((_LANES,), count)

    def step2(j, carry):
        k = j * _LANES
        mf = m_v[pl.ds(k, _LANES)]
        m_v[pl.ds(k, _LANES)] = jnp.where(mf > 0.5, h16, 0.0)
        return carry

    lax.fori_loop(0, _ITERS, step2, 0, unroll=4)

    ocp1 = pltpu.async_copy(v_v, v_hbm.at[pl.ds(base, _HALF)], sem)
    ocp2 = pltpu.async_copy(m_v, h_hbm.at[pl.ds(base, _HALF)], sem)

    @pl.when(jnp.logical_and(c == 0, s == 0))
    def _():
        rr = r_v[...]
        lam_v[...] = thresh_vec * 0.5 / (rr * rr)
        pltpu.sync_copy(lam_v, lam_hbm)

    ocp1.wait()
    ocp2.wait()


_mesh = plsc.VectorSubcoreMesh(
    core_axis_name="c", subcore_axis_name="s", num_cores=2, num_subcores=16)

_sc_call = functools.partial(
    pl.kernel,
    out_type=[
        jax.ShapeDtypeStruct((_N,), jnp.float32),
        jax.ShapeDtypeStruct((_N,), jnp.float32),
        jax.ShapeDtypeStruct((_B,), jnp.float32),
    ],
    mesh=_mesh,
    compiler_params=pltpu.CompilerParams(needs_layout_passes=False),
    scratch_types=[
        pltpu.VMEM((_HALF,), jnp.float32),
        pltpu.VMEM((_HALF,), jnp.float32),
        pltpu.VMEM((_HALF,), jnp.float32),
        pltpu.VMEM((_HALF,), jnp.float32),
        pltpu.VMEM((_HALF,), jnp.float32),
        pltpu.VMEM((_HALF,), jnp.float32),
        pltpu.VMEM((_HALF,), jnp.float32),
        pltpu.VMEM((_HALF,), jnp.float32),
        pltpu.VMEM((_LANES,), jnp.float32),
        pltpu.VMEM((_LANES,), jnp.float32),
        pltpu.VMEM((_LANES,), jnp.float32),
        pltpu.VMEM((_LANES,), jnp.float32),
        pltpu.VMEM((_LANES,), jnp.float32),
        pltpu.VMEM((_LANES,), jnp.float32),
        pltpu.VMEM((_LANES,), jnp.float32),
        pltpu.VMEM_SHARED((16 * _LANES,), jnp.float32),
        pltpu.SemaphoreType.DMA,
    ],
)(_body)


@jax.jit
def kernel(u_pred, num_hits, R, mag_pred, sigma_mag, u_obs, thresh_s2_param):
    del mag_pred, sigma_mag
    upt = u_pred.T.reshape(-1)
    uot = u_obs.T.reshape(-1)
    lr = jnp.asarray(_LOG_RANGE)
    v, h_vec, lam = _sc_call(upt, uot, num_hits, R, thresh_s2_param, lr)
    return v, h_vec, lam


# speedup vs baseline: 1.0148x; 1.0148x over previous
import functools

import jax
import jax.numpy as jnp
import numpy as np
from jax import lax
from jax.experimental import pallas as pl
from jax.experimental.pallas import tpu as pltpu
from jax.experimental.pallas import tpu_sc as plsc

_B = 16
_ROW = 2048
_N = _B * _ROW
_LANES = 16
_HALF = _ROW // 2
_ITERS = _HALF // _LANES


def _deg2dist(deg):
    return 2.0 * np.sin(np.radians(deg) / 2.0)


_THRESH_DEG = np.ones(_B, dtype=np.float32)
_T_MIN = np.float32(_deg2dist(10.0 / 3600.0) ** 2)
_T_MAX = (_deg2dist(_THRESH_DEG) ** 2).astype(np.float32)
_LOG_RANGE = np.log(_T_MAX / _T_MIN).astype(np.float32)
_INV_T_MIN = float(1.0 / _T_MIN)


def _body(up_hbm, uo_hbm, nh_hbm, r_hbm, pp_hbm, lr_hbm,
          v_hbm, h_hbm, lam_hbm,
          ux_v, uy_v, uz_v, ox_v, oy_v, oz_v, v_v, m_v,
          nh_v, r_v, pp_v, lr_v, lam_v, cnt_v, pc_v, shared, sem):
    c = lax.axis_index("c")
    s = lax.axis_index("s")
    b = c * 8 + lax.div(s, 2)
    half = lax.rem(s, 2)
    base = b * _ROW + half * _HALF

    cps = [
        pltpu.async_copy(up_hbm.at[pl.ds(base, _HALF)], ux_v, sem),
        pltpu.async_copy(up_hbm.at[pl.ds(_N + base, _HALF)], uy_v, sem),
        pltpu.async_copy(up_hbm.at[pl.ds(2 * _N + base, _HALF)], uz_v, sem),
        pltpu.async_copy(uo_hbm.at[pl.ds(base, _HALF)], ox_v, sem),
        pltpu.async_copy(uo_hbm.at[pl.ds(_N + base, _HALF)], oy_v, sem),
        pltpu.async_copy(uo_hbm.at[pl.ds(2 * _N + base, _HALF)], oz_v, sem),
        pltpu.async_copy(nh_hbm, nh_v, sem),
        pltpu.async_copy(r_hbm, r_v, sem),
        pltpu.async_copy(pp_hbm, pp_v, sem),
        pltpu.async_copy(lr_hbm, lr_v, sem),
    ]
    for cp in cps:
        cp.wait()

    lane = lax.iota(jnp.int32, _LANES)
    sel = lane == b

    pl_ = pp_v[...] * lr_v[...]
    thresh_vec = _T_MIN * jnp.exp(pl_)
    inv_thresh_vec = _INV_T_MIN * jnp.exp(-pl_)
    thr = jnp.sum(jnp.where(sel, thresh_vec, 0.0))
    inv_thr = jnp.sum(jnp.where(sel, inv_thresh_vec, 0.0))
    nh = jnp.sum(jnp.where(sel, nh_v[...], 0.0))

    def step(j, cnt):
        k = j * _LANES
        sl = pl.ds(k, _LANES)
        dx = ux_v[sl] - ox_v[sl]
        dy = uy_v[sl] - oy_v[sl]
        dz = uz_v[sl] - oz_v[sl]
        s2 = dx * dx + dy * dy + dz * dz
        m = s2 < thr
        mf = jnp.where(m, 1.0, 0.0)
        v_v[sl] = jnp.where(m, s2 * inv_thr, 0.0)
        m_v[sl] = mf
        return cnt + mf

    cnt = lax.fori_loop(0, _ITERS, step, jnp.zeros((_LANES,), jnp.float32))

    cnt_v[...] = cnt
    pltpu.sync_copy(cnt_v, shared.at[pl.ds(s * _LANES, _LANES)])
    plsc.subcore_barrier()
    partner = s + 1 - 2 * half
    pltpu.sync_copy(shared.at[pl.ds(partner * _LANES, _LANES)], pc_v)
    count = jnp.sum(cnt + pc_v[...])
    h16 = jnp.full((_LANES,), nh) / jnp.full((_LANES,), count)

    def step2(j, carry):
        k = j * _LANES
        mf = m_v[pl.ds(k, _LANES)]
        m_v[pl.ds(k, _LANES)] = jnp.where(mf > 0.5, h16, 0.0)
        return carry

    lax.fori_loop(0, _ITERS, step2, 0)

    ocp1 = pltpu.async_copy(v_v, v_hbm.at[pl.ds(base, _HALF)], sem)
    ocp2 = pltpu.async_copy(m_v, h_hbm.at[pl.ds(base, _HALF)], sem)

    @pl.when(jnp.logical_and(c == 0, s == 0))
    def _():
        rr = r_v[...]
        lam_v[...] = thresh_vec * 0.5 / (rr * rr)
        pltpu.sync_copy(lam_v, lam_hbm)

    ocp1.wait()
    ocp2.wait()


_mesh = plsc.VectorSubcoreMesh(
    core_axis_name="c", subcore_axis_name="s", num_cores=2, num_subcores=16)

_sc_call = functools.partial(
    pl.kernel,
    out_type=[
        jax.ShapeDtypeStruct((_N,), jnp.float32),
        jax.ShapeDtypeStruct((_N,), jnp.float32),
        jax.ShapeDtypeStruct((_B,), jnp.float32),
    ],
    mesh=_mesh,
    compiler_params=pltpu.CompilerParams(needs_layout_passes=False),
    scratch_types=[
        pltpu.VMEM((_HALF,), jnp.float32),
        pltpu.VMEM((_HALF,), jnp.float32),
        pltpu.VMEM((_HALF,), jnp.float32),
        pltpu.VMEM((_HALF,), jnp.float32),
        pltpu.VMEM((_HALF,), jnp.float32),
        pltpu.VMEM((_HALF,), jnp.float32),
        pltpu.VMEM((_HALF,), jnp.float32),
        pltpu.VMEM((_HALF,), jnp.float32),
        pltpu.VMEM((_LANES,), jnp.float32),
        pltpu.VMEM((_LANES,), jnp.float32),
        pltpu.VMEM((_LANES,), jnp.float32),
        pltpu.VMEM((_LANES,), jnp.float32),
        pltpu.VMEM((_LANES,), jnp.float32),
        pltpu.VMEM((_LANES,), jnp.float32),
        pltpu.VMEM((_LANES,), jnp.float32),
        pltpu.VMEM_SHARED((16 * _LANES,), jnp.float32),
        pltpu.SemaphoreType.DMA,
    ],
)(_body)


@jax.jit
def kernel(u_pred, num_hits, R, mag_pred, sigma_mag, u_obs, thresh_s2_param):
    del mag_pred, sigma_mag
    upt = u_pred.T.reshape(-1)
    uot = u_obs.T.reshape(-1)
    lr = jnp.asarray(_LOG_RANGE)
    v, h_vec, lam = _sc_call(upt, uot, num_hits, R, thresh_s2_param, lr)
    return v, h_vec, lam


# speedup vs baseline: 1.0428x; 1.0276x over previous
import functools

import jax
import jax.numpy as jnp
import numpy as np
from jax import lax
from jax.experimental import pallas as pl
from jax.experimental.pallas import tpu as pltpu
from jax.experimental.pallas import tpu_sc as plsc

_B = 16
_ROW = 2048
_N = _B * _ROW
_LANES = 16
_HALF = _ROW // 2
_ITERS = _HALF // _LANES


def _deg2dist(deg):
    return 2.0 * np.sin(np.radians(deg) / 2.0)


_THRESH_DEG = np.ones(_B, dtype=np.float32)
_T_MIN = np.float32(_deg2dist(10.0 / 3600.0) ** 2)
_T_MAX = (_deg2dist(_THRESH_DEG) ** 2).astype(np.float32)
_LOG_RANGE = np.log(_T_MAX / _T_MIN).astype(np.float32)
_INV_T_MIN = float(1.0 / _T_MIN)
_LOG_RANGE_C = float(_LOG_RANGE[0])


def _body(up_hbm, uo_hbm, nh_hbm, r_hbm, pp_hbm,
          v_hbm, h_hbm, lam_hbm,
          ux_v, uy_v, uz_v, ox_v, oy_v, oz_v, v_v, m_v,
          nh_v, r_v, pp_v, lam_v, cnt_v, pc_v, shared, sem):
    c = lax.axis_index("c")
    s = lax.axis_index("s")
    b = c * 8 + lax.div(s, 2)
    half = lax.rem(s, 2)
    base = b * _ROW + half * _HALF

    cps = [
        pltpu.async_copy(up_hbm.at[pl.ds(base, _HALF)], ux_v, sem),
        pltpu.async_copy(up_hbm.at[pl.ds(_N + base, _HALF)], uy_v, sem),
        pltpu.async_copy(up_hbm.at[pl.ds(2 * _N + base, _HALF)], uz_v, sem),
        pltpu.async_copy(uo_hbm.at[pl.ds(base, _HALF)], ox_v, sem),
        pltpu.async_copy(uo_hbm.at[pl.ds(_N + base, _HALF)], oy_v, sem),
        pltpu.async_copy(uo_hbm.at[pl.ds(2 * _N + base, _HALF)], oz_v, sem),
        pltpu.async_copy(nh_hbm, nh_v, sem),
        pltpu.async_copy(r_hbm, r_v, sem),
        pltpu.async_copy(pp_hbm, pp_v, sem),
    ]
    for cp in cps:
        cp.wait()

    lane = lax.iota(jnp.int32, _LANES)
    sel = lane == b

    pl_ = pp_v[...] * _LOG_RANGE_C
    thresh_vec = _T_MIN * jnp.exp(pl_)
    inv_thresh_vec = _INV_T_MIN * jnp.exp(-pl_)
    thr = jnp.sum(jnp.where(sel, thresh_vec, 0.0))
    inv_thr = jnp.sum(jnp.where(sel, inv_thresh_vec, 0.0))
    nh = jnp.sum(jnp.where(sel, nh_v[...], 0.0))

    def step(j, cnt):
        k = j * _LANES
        sl = pl.ds(k, _LANES)
        dx = ux_v[sl] - ox_v[sl]
        dy = uy_v[sl] - oy_v[sl]
        dz = uz_v[sl] - oz_v[sl]
        s2 = dx * dx + dy * dy + dz * dz
        m = s2 < thr
        mf = jnp.where(m, 1.0, 0.0)
        v_v[sl] = jnp.where(m, s2 * inv_thr, 0.0)
        m_v[sl] = mf
        return cnt + mf

    cnt = lax.fori_loop(0, _ITERS, step, jnp.zeros((_LANES,), jnp.float32))

    cnt_v[...] = cnt
    pltpu.sync_copy(cnt_v, shared.at[pl.ds(s * _LANES, _LANES)])
    plsc.subcore_barrier()
    partner = s + 1 - 2 * half
    pltpu.sync_copy(shared.at[pl.ds(partner * _LANES, _LANES)], pc_v)
    count = jnp.sum(cnt + pc_v[...])
    h16 = jnp.full((_LANES,), nh) / jnp.full((_LANES,), count)

    def step2(j, carry):
        k = j * _LANES
        mf = m_v[pl.ds(k, _LANES)]
        m_v[pl.ds(k, _LANES)] = jnp.where(mf > 0.5, h16, 0.0)
        return carry

    lax.fori_loop(0, _ITERS, step2, 0)

    ocp1 = pltpu.async_copy(v_v, v_hbm.at[pl.ds(base, _HALF)], sem)
    ocp2 = pltpu.async_copy(m_v, h_hbm.at[pl.ds(base, _HALF)], sem)

    @pl.when(jnp.logical_and(c == 0, s == 0))
    def _():
        rr = r_v[...]
        lam_v[...] = thresh_vec * 0.5 / (rr * rr)
        pltpu.sync_copy(lam_v, lam_hbm)

    ocp1.wait()
    ocp2.wait()


_mesh = plsc.VectorSubcoreMesh(
    core_axis_name="c", subcore_axis_name="s", num_cores=2, num_subcores=16)

_sc_call = functools.partial(
    pl.kernel,
    out_type=[
        jax.ShapeDtypeStruct((_N,), jnp.float32),
        jax.ShapeDtypeStruct((_N,), jnp.float32),
        jax.ShapeDtypeStruct((_B,), jnp.float32),
    ],
    mesh=_mesh,
    compiler_params=pltpu.CompilerParams(needs_layout_passes=False),
    scratch_types=[
        pltpu.VMEM((_HALF,), jnp.float32),
        pltpu.VMEM((_HALF,), jnp.float32),
        pltpu.VMEM((_HALF,), jnp.float32),
        pltpu.VMEM((_HALF,), jnp.float32),
        pltpu.VMEM((_HALF,), jnp.float32),
        pltpu.VMEM((_HALF,), jnp.float32),
        pltpu.VMEM((_HALF,), jnp.float32),
        pltpu.VMEM((_HALF,), jnp.float32),
        pltpu.VMEM((_LANES,), jnp.float32),
        pltpu.VMEM((_LANES,), jnp.float32),
        pltpu.VMEM((_LANES,), jnp.float32),
        pltpu.VMEM((_LANES,), jnp.float32),
        pltpu.VMEM((_LANES,), jnp.float32),
        pltpu.VMEM((_LANES,), jnp.float32),
        pltpu.VMEM_SHARED((16 * _LANES,), jnp.float32),
        pltpu.SemaphoreType.DMA,
    ],
)(_body)


@jax.jit
def kernel(u_pred, num_hits, R, mag_pred, sigma_mag, u_obs, thresh_s2_param):
    del mag_pred, sigma_mag
    upt = u_pred.T.reshape(-1)
    uot = u_obs.T.reshape(-1)
    v, h_vec, lam = _sc_call(upt, uot, num_hits, R, thresh_s2_param)
    return v, h_vec, lam
